# baseline (speedup 1.0000x reference)
import functools

import jax
import jax.numpy as jnp
from jax import lax
from jax.experimental import pallas as pl
from jax.experimental.pallas import tpu as pltpu
from jax.experimental.pallas import tpu_sc as plsc

L = 16
NS = 16

_LN2_HI = 0.693359375
_LN2_LO = -2.12194440e-4
_SQRT2_BITS = 0x3fb504f3


def _log16(x):
    bits = plsc.bitcast(x, jnp.int32)
    e = lax.shift_right_logical(bits, 23) - 127
    m_bits = (bits & 0x007FFFFF) | 0x3F800000
    m = plsc.bitcast(m_bits, jnp.float32)
    big = m_bits >= _SQRT2_BITS
    m = jnp.where(big, m * 0.5, m)
    e = (e + big.astype(jnp.int32)).astype(jnp.float32)
    f = m - 1.0
    z = f * f
    p = jnp.float32(7.0376836292e-2)
    p = p * f + -1.1514610310e-1
    p = p * f + 1.1676998740e-1
    p = p * f + -1.2420140846e-1
    p = p * f + 1.4249322787e-1
    p = p * f + -1.6668057665e-1
    p = p * f + 2.0000714765e-1
    p = p * f + -2.4999993993e-1
    p = p * f + 3.3333331174e-1
    y = f * z * p
    y = y + e * _LN2_LO
    y = y - 0.5 * z
    return f + y + e * _LN2_HI


def _make_sc_kernel(B, V):
    n = B // NS
    nv = n // L
    mesh = plsc.VectorSubcoreMesh(core_axis_name="c", subcore_axis_name="s",
                                  num_cores=1)

    @functools.partial(
        pl.kernel,
        out_type=jax.ShapeDtypeStruct((NS, L), jnp.float32),
        mesh=mesh,
        scratch_types=[
            pltpu.VMEM((n,), jnp.int32),
            pltpu.VMEM((n,), jnp.int32),
            pltpu.VMEM((n,), jnp.float32),
            pltpu.VMEM((L,), jnp.float32),
            pltpu.SemaphoreType.DMA,
        ],
        compiler_params=pltpu.CompilerParams(needs_layout_passes=False),
    )
    def sc_loss(flat_hbm, tgt_hbm, out_hbm,
                tgt_v, idx_v, vals_v, stage_v, sem):
        sid = lax.axis_index("s")
        base = sid * n
        pltpu.sync_copy(tgt_hbm.at[pl.ds(base, n)], tgt_v)
        for j in range(nv):
            t = tgt_v[pl.ds(j * L, L)]
            row = base + j * L + lax.iota(jnp.int32, L)
            idx_v[pl.ds(j * L, L)] = (
                lax.shift_right_logical(t, 3) * 8192
                + lax.shift_right_logical(row, 7) * 1024
                + (t & 7) * 128
                + (row & 127)
            )
        pltpu.async_copy(flat_hbm.at[idx_v], vals_v, sem).wait()
        acc = _log16(vals_v[pl.ds(0, L)])
        for j in range(1, nv):
            acc = acc + _log16(vals_v[pl.ds(j * L, L)])
        acc = acc * (-1.0 / B)
        stage_v[...] = acc
        pltpu.sync_copy(stage_v, out_hbm.at[sid])

    return sc_loss


def kernel(input, target):
    B, V = input.shape
    perm = jnp.transpose(input.reshape(B // 128, 128, V // 8, 8), (2, 0, 3, 1))
    flat = perm.reshape(B * V)
    tgt = target.astype(jnp.int32)
    part = _make_sc_kernel(B, V)(flat, tgt)
    return jnp.sum(part)


# speedup vs baseline: 1.0222x; 1.0222x over previous
import functools

import jax
import jax.numpy as jnp
from jax import lax
from jax.experimental import pallas as pl
from jax.experimental.pallas import tpu as pltpu
from jax.experimental.pallas import tpu_sc as plsc

L = 16
NS = 16

_LN2_HI = 0.693359375
_LN2_LO = -2.12194440e-4
_SQRT2_BITS = 0x3fb504f3


def _log16(x):
    bits = plsc.bitcast(x, jnp.int32)
    e = lax.shift_right_logical(bits, 23) - 127
    m_bits = (bits & 0x007FFFFF) | 0x3F800000
    m = plsc.bitcast(m_bits, jnp.float32)
    big = m_bits >= _SQRT2_BITS
    m = jnp.where(big, m * 0.5, m)
    e = (e + big.astype(jnp.int32)).astype(jnp.float32)
    f = m - 1.0
    z = f * f
    p = jnp.float32(7.0376836292e-2)
    p = p * f + -1.1514610310e-1
    p = p * f + 1.1676998740e-1
    p = p * f + -1.2420140846e-1
    p = p * f + 1.4249322787e-1
    p = p * f + -1.6668057665e-1
    p = p * f + 2.0000714765e-1
    p = p * f + -2.4999993993e-1
    p = p * f + 3.3333331174e-1
    y = f * z * p
    y = y + e * _LN2_LO
    y = y - 0.5 * z
    return f + y + e * _LN2_HI


def _make_sc_kernel(B, V):
    n = B // NS
    nv = n // L
    mesh = plsc.VectorSubcoreMesh(core_axis_name="c", subcore_axis_name="s",
                                  num_cores=1)

    @functools.partial(
        pl.kernel,
        out_type=[
            jax.ShapeDtypeStruct((NS, L), jnp.float32),
            jax.ShapeDtypeStruct((L,), jnp.float32),
        ],
        mesh=mesh,
        scratch_types=[
            pltpu.VMEM((n,), jnp.int32),
            pltpu.VMEM((n,), jnp.int32),
            pltpu.VMEM((n,), jnp.float32),
            pltpu.VMEM((L,), jnp.float32),
            pltpu.VMEM((NS, L), jnp.float32),
            pltpu.SemaphoreType.DMA,
        ],
        compiler_params=pltpu.CompilerParams(needs_layout_passes=False, disable_bounds_checks=True, disable_semaphore_checks=True),
    )
    def sc_loss(flat_hbm, tgt_hbm, part_hbm, out_hbm,
                tgt_v, idx_v, vals_v, stage_v, all_v, sem):
        cid = lax.axis_index("c")
        sid = lax.axis_index("s")
        base = sid * n
        pltpu.sync_copy(tgt_hbm.at[pl.ds(base, n)], tgt_v)
        for j in range(nv):
            t = tgt_v[pl.ds(j * L, L)]
            row = base + j * L + lax.iota(jnp.int32, L)
            idx_v[pl.ds(j * L, L)] = (
                lax.shift_right_logical(t, 3) * 8192
                + lax.shift_right_logical(row, 7) * 1024
                + (t & 7) * 128
                + (row & 127)
            )
        pltpu.async_copy(flat_hbm.at[idx_v], vals_v, sem).wait()
        acc = _log16(vals_v[pl.ds(0, L)])
        for j in range(1, nv):
            acc = acc + _log16(vals_v[pl.ds(j * L, L)])
        stage_v[...] = acc

        @pl.when(cid == 0)
        def _():
            pltpu.sync_copy(stage_v, part_hbm.at[sid])

        plsc.subcore_barrier()

        @pl.when((sid == 0) & (cid == 0))
        def _():
            pltpu.sync_copy(part_hbm, all_v)
            tot = all_v[0]
            for k in range(1, NS):
                tot = tot + all_v[k]
            s = jnp.sum(tot)
            stage_v[...] = jnp.full((L,), s * (-1.0 / B), jnp.float32)
            pltpu.sync_copy(stage_v, out_hbm)

    return sc_loss


def kernel(input, target):
    B, V = input.shape
    perm = jnp.transpose(input.reshape(B // 128, 128, V // 8, 8), (2, 0, 3, 1))
    flat = perm.reshape(B * V)
    tgt = target.astype(jnp.int32)
    _, out = _make_sc_kernel(B, V)(flat, tgt)
    return out[0]
